# baseline (speedup 1.0000x reference)
import functools

import jax
import jax.numpy as jnp
from jax import lax
from jax.experimental import pallas as pl
from jax.experimental.pallas import tpu as pltpu
from jax.experimental.pallas import tpu_sc as plsc

N = 10000
E = 320000
NC = 2
NS = 16
NW = NC * NS
CHUNK = 80
EPT = E // NW
NCHUNK = EPT // CHUNK
ROWS_T = 624
ROWS_LAST = N - (NS - 1) * ROWS_T
NB = 5


def _mesh():
    return plsc.VectorSubcoreMesh(core_axis_name="c", subcore_axis_name="s")


_SC_PARAMS = pltpu.CompilerParams(use_tc_tiling_on_sc=False)
_SC_PARAMS_NOLAYOUT = pltpu.CompilerParams(use_tc_tiling_on_sc=False,
                                           needs_layout_passes=False)


def _zero_acc(obuf, acc, s, nrow_cols):
    r0 = s * ROWS_T
    if len(nrow_cols) == 1:
        def zrow(i, carry):
            obuf[pl.ds(i * 16, 16)] = jnp.zeros((16,), jnp.float32)
            return carry
        lax.fori_loop(0, ROWS_LAST // 16, zrow, None)
    else:
        F = nrow_cols[1]

        def zrow(i, carry):
            for j in range(F // 16):
                obuf[i, pl.ds(j * 16, 16)] = jnp.zeros((16,), jnp.float32)
            return carry
        lax.fori_loop(0, ROWS_LAST, zrow, None)

    @pl.when(s < NS - 1)
    def _z0():
        pltpu.sync_copy(obuf.at[pl.ds(0, ROWS_T)], acc.at[pl.ds(r0, ROWS_T)])

    @pl.when(s == NS - 1)
    def _z1():
        pltpu.sync_copy(obuf, acc.at[pl.ds(r0, ROWS_LAST)])


def _writeout(obuf, acc, oref, s):
    r0 = s * ROWS_T

    @pl.when(s < NS - 1)
    def _w0():
        pltpu.sync_copy(acc.at[pl.ds(r0, ROWS_T)], obuf.at[pl.ds(0, ROWS_T)])
        pltpu.sync_copy(obuf.at[pl.ds(0, ROWS_T)], oref.at[pl.ds(r0, ROWS_T)])

    @pl.when(s == NS - 1)
    def _w1():
        pltpu.sync_copy(acc.at[pl.ds(r0, ROWS_LAST)], obuf)
        pltpu.sync_copy(obuf, oref.at[pl.ds(r0, ROWS_LAST)])


def _deg_partials(dst):
    @functools.partial(
        pl.kernel,
        mesh=_mesh(),
        compiler_params=_SC_PARAMS_NOLAYOUT,
        out_type=[jax.ShapeDtypeStruct((N,), jnp.float32),
                  jax.ShapeDtypeStruct((N,), jnp.float32)],
        scratch_types=[
            pltpu.VMEM((EPT,), jnp.int32),
            pltpu.VMEM((N,), jnp.float32),
            pltpu.VMEM((NS, ROWS_LAST), jnp.float32),
            pltpu.VMEM((ROWS_LAST,), jnp.float32),
            pltpu.VMEM_SHARED((NS, N), jnp.float32),
        ],
    )
    def k(dst_hbm, out0_hbm, out1_hbm, didx, hist, buf, sbuf, slab):
        c = lax.axis_index("c")
        s = lax.axis_index("s")
        wid = c * NS + s
        pltpu.sync_copy(dst_hbm.at[pl.ds(wid * EPT, EPT)], didx)

        def zh(i, carry):
            hist[pl.ds(i * 16, 16)] = jnp.zeros((16,), jnp.float32)
            return carry

        lax.fori_loop(0, N // 16, zh, None)
        ones16 = jnp.ones((16,), jnp.float32)

        def hb(i, carry):
            idxv = didx[pl.ds(i * 16, 16)]
            plsc.addupdate_scatter(hist, [idxv], ones16)
            return carry

        lax.fori_loop(0, EPT // 16, hb, None)
        pltpu.sync_copy(hist, slab.at[s])
        plsc.subcore_barrier()

        r0 = s * ROWS_T

        @pl.when(s < NS - 1)
        def _l0():
            for r in range(NS):
                pltpu.sync_copy(slab.at[r, pl.ds(r0, ROWS_T)],
                                buf.at[r, pl.ds(0, ROWS_T)])

        @pl.when(s == NS - 1)
        def _l1():
            for r in range(NS):
                pltpu.sync_copy(slab.at[r, pl.ds(r0, ROWS_LAST)], buf.at[r])

        def sb(kk, carry):
            acc16 = buf[0, pl.ds(kk * 16, 16)]
            for r in range(1, NS):
                acc16 = acc16 + buf[r, pl.ds(kk * 16, 16)]
            sbuf[pl.ds(kk * 16, 16)] = acc16
            return carry

        lax.fori_loop(0, ROWS_LAST // 16, sb, None)

        for ci, oref in ((0, out0_hbm), (1, out1_hbm)):
            @pl.when((c == ci) & (s < NS - 1))
            def _w0(oref=oref):
                pltpu.sync_copy(sbuf.at[pl.ds(0, ROWS_T)],
                                oref.at[pl.ds(r0, ROWS_T)])

            @pl.when((c == ci) & (s == NS - 1))
            def _w1(oref=oref):
                pltpu.sync_copy(sbuf, oref.at[pl.ds(r0, ROWS_LAST)])

    return k(dst)


def _agg_partials(g, src, dst, F):
    @functools.partial(
        pl.kernel,
        mesh=_mesh(),
        compiler_params=_SC_PARAMS,
        out_type=jax.ShapeDtypeStruct((NC, N, F), jnp.float32),
        scratch_types=[
            pltpu.VMEM((EPT,), jnp.int32),
            pltpu.VMEM((EPT,), jnp.int32),
            [pltpu.VMEM((CHUNK, F), jnp.float32)] * NB,
            pltpu.VMEM((ROWS_LAST, F), jnp.float32),
            pltpu.VMEM_SHARED((N, F), jnp.float32),
            [pltpu.SemaphoreType.DMA] * NB,
            [pltpu.SemaphoreType.DMA] * NB,
        ],
    )
    def k(g_hbm, src_hbm, dst_hbm, out_hbm, sidx, didx, rows, obuf,
          acc, gsems, ssems):
        c = lax.axis_index("c")
        s = lax.axis_index("s")
        wid = c * NS + s
        pltpu.sync_copy(src_hbm.at[pl.ds(wid * EPT, EPT)], sidx)
        pltpu.sync_copy(dst_hbm.at[pl.ds(wid * EPT, EPT)], didx)
        _zero_acc(obuf, acc, s, (N, F))
        plsc.subcore_barrier()

        def sl(ref, i):
            return ref.at[pl.ds(i * CHUNK, CHUNK)]

        for b in range(NB):
            pltpu.async_copy(g_hbm.at[sl(sidx, b)], rows[b], gsems[b])

        def body(t, carry):
            for b in range(NB):
                i = t * NB + b
                pltpu.make_async_copy(g_hbm.at[sl(sidx, i)], rows[b],
                                      gsems[b]).wait()
                pltpu.async_copy(rows[b], acc.at[sl(didx, i)], ssems[b],
                                 add=True)
                pltpu.make_async_copy(rows[b], acc.at[sl(didx, i)],
                                      ssems[b]).wait()
                pltpu.async_copy(g_hbm.at[sl(sidx, i + NB)], rows[b], gsems[b])
            return carry

        lax.fori_loop(0, NCHUNK // NB - 1, body, None)
        for b in range(NB):
            i = NCHUNK - NB + b
            pltpu.make_async_copy(g_hbm.at[sl(sidx, i)], rows[b],
                                  gsems[b]).wait()
            pltpu.async_copy(rows[b], acc.at[sl(didx, i)], ssems[b], add=True)
            pltpu.make_async_copy(rows[b], acc.at[sl(didx, i)],
                                  ssems[b]).wait()
        plsc.subcore_barrier()
        _writeout(obuf, acc, out_hbm.at[c], s)

    return k(g, src, dst)


def _edge_scores(h3, ei, ej):
    @functools.partial(
        pl.kernel,
        mesh=_mesh(),
        compiler_params=_SC_PARAMS_NOLAYOUT,
        out_type=jax.ShapeDtypeStruct((E,), jnp.float32),
        scratch_types=[
            pltpu.VMEM((EPT,), jnp.int32),
            pltpu.VMEM((EPT,), jnp.int32),
            [pltpu.VMEM((CHUNK, 64), jnp.float32)] * NB,
            [pltpu.VMEM((CHUNK, 64), jnp.float32)] * NB,
            [pltpu.VMEM((CHUNK,), jnp.float32)] * NB,
            pltpu.VMEM((16, 17), jnp.float32),
            [pltpu.SemaphoreType.DMA] * NB,
            [pltpu.SemaphoreType.DMA] * NB,
        ],
    )
    def k(h_hbm, ei_hbm, ej_hbm, out_hbm, ii, jj, ri, rj, sv, pbuf,
          gsems, wsems):
        c = lax.axis_index("c")
        s = lax.axis_index("s")
        wid = c * NS + s
        base = wid * EPT
        pltpu.sync_copy(ei_hbm.at[pl.ds(wid * EPT, EPT)], ii)
        pltpu.sync_copy(ej_hbm.at[pl.ds(wid * EPT, EPT)], jj)

        def sl(ref, i):
            return ref.at[pl.ds(i * CHUNK, CHUNK)]

        def start_gathers(i, b):
            pltpu.async_copy(h_hbm.at[sl(ii, i)], ri[b], gsems[b])
            pltpu.async_copy(h_hbm.at[sl(jj, i)], rj[b], gsems[b])

        def wait_gathers(i, b):
            pltpu.make_async_copy(h_hbm.at[sl(ii, i)], ri[b], gsems[b]).wait()
            pltpu.make_async_copy(h_hbm.at[sl(jj, i)], rj[b], gsems[b]).wait()

        def compute(b):
            def grp_body(grp, carry):
                def e_body(u4, carry2):
                    for u in range(4):
                        e = grp * 16 + u4 * 4 + u
                        p = (ri[b][e, pl.ds(0, 16)] * rj[b][e, pl.ds(0, 16)])
                        for f in range(1, 4):
                            p = p + (ri[b][e, pl.ds(f * 16, 16)]
                                     * rj[b][e, pl.ds(f * 16, 16)])
                        pbuf[u4 * 4 + u, pl.ds(0, 16)] = p
                    return carry2

                lax.fori_loop(0, 4, e_body, None)
                rvec = jnp.zeros((16,), jnp.float32)
                urow = lax.iota(jnp.int32, 16)
                for f in range(16):
                    cf = jnp.full((16,), f, jnp.int32)
                    rvec = rvec + plsc.load_gather(pbuf, [urow, cf])
                sv[b][pl.ds(grp * 16, 16)] = rvec
                return carry

            lax.fori_loop(0, CHUNK // 16, grp_body, None)

        def start_write(i, b):
            pltpu.async_copy(sv[b], out_hbm.at[pl.ds(base + i * CHUNK, CHUNK)],
                             wsems[b])

        def wait_write(i, b):
            pltpu.make_async_copy(sv[b],
                                  out_hbm.at[pl.ds(base + i * CHUNK, CHUNK)],
                                  wsems[b]).wait()

        for b in range(NB):
            start_gathers(b, b)

        def body(t, carry):
            for b in range(NB):
                i = t * NB + b
                wait_gathers(i, b)

                @pl.when(t > 0)
                def _ww():
                    wait_write(i - NB, b)

                compute(b)
                start_write(i, b)
                start_gathers(i + NB, b)
            return carry

        lax.fori_loop(0, NCHUNK // NB - 1, body, None)
        for b in range(NB):
            i = NCHUNK - NB + b
            wait_gathers(i, b)
            wait_write(i - NB, b)
            compute(b)
            start_write(i, b)
        for b in range(NB):
            wait_write(NCHUNK - NB + b, b)

    return k(h3, ei, ej)


BR = 2000
GRID = N // BR


def _tc_first(degp, x, W1):
    def body(degp_ref, x_ref, w_ref, dinv_ref, g_ref):
        deg = degp_ref[:, 0:1] + degp_ref[:, 1:2] + 1.0
        dinv = lax.rsqrt(deg)
        dinv_ref[...] = dinv
        g = dinv * jnp.dot(x_ref[...], w_ref[...],
                           preferred_element_type=jnp.float32)
        g_ref[0] = g[:, :64]
        g_ref[1] = g[:, 64:]

    return pl.pallas_call(
        body,
        grid=(GRID,),
        in_specs=[
            pl.BlockSpec((BR, 2), lambda i: (i, 0)),
            pl.BlockSpec((BR, 128), lambda i: (i, 0)),
            pl.BlockSpec((128, 128), lambda i: (0, 0)),
        ],
        out_specs=[
            pl.BlockSpec((BR, 1), lambda i: (i, 0)),
            pl.BlockSpec((2, BR, 64), lambda i: (0, i, 0)),
        ],
        out_shape=[
            jax.ShapeDtypeStruct((N, 1), jnp.float32),
            jax.ShapeDtypeStruct((2, N, 64), jnp.float32),
        ],
    )(degp, x, W1)


def _tc_layer1(pa, pb, ga, gb, dinv, b, Wn):
    def body(pa_ref, pb_ref, ga_ref, gb_ref, dinv_ref, b_ref, w_ref, gn_ref):
        sa = pa_ref[0] + pa_ref[1] + ga_ref[...]
        sb = pb_ref[0] + pb_ref[1] + gb_ref[...]
        s = jnp.concatenate([sa, sb], axis=1)
        h = jnp.maximum(dinv_ref[...] * s + b_ref[...], 0.0)
        gn_ref[...] = dinv_ref[...] * jnp.dot(h, w_ref[...],
                                              preferred_element_type=jnp.float32)

    return pl.pallas_call(
        body,
        grid=(GRID,),
        in_specs=[
            pl.BlockSpec((NC, BR, 64), lambda i: (0, i, 0)),
            pl.BlockSpec((NC, BR, 64), lambda i: (0, i, 0)),
            pl.BlockSpec((BR, 64), lambda i: (i, 0)),
            pl.BlockSpec((BR, 64), lambda i: (i, 0)),
            pl.BlockSpec((BR, 1), lambda i: (i, 0)),
            pl.BlockSpec((1, 128), lambda i: (0, 0)),
            pl.BlockSpec((128, 64), lambda i: (0, 0)),
        ],
        out_specs=pl.BlockSpec((BR, 64), lambda i: (i, 0)),
        out_shape=jax.ShapeDtypeStruct((N, 64), jnp.float32),
    )(pa, pb, ga, gb, dinv, b, Wn)


def _tc_layer(p, g, dinv, b, Wn, Fin, Fout):
    def body(p_ref, g_ref, dinv_ref, b_ref, w_ref, h_ref, gn_ref):
        pr = p_ref[0] + p_ref[1]
        h = jnp.maximum(dinv_ref[...] * (pr + g_ref[...]) + b_ref[...], 0.0)
        h_ref[...] = h
        gn_ref[...] = dinv_ref[...] * jnp.dot(h, w_ref[...],
                                              preferred_element_type=jnp.float32)

    return pl.pallas_call(
        body,
        grid=(GRID,),
        in_specs=[
            pl.BlockSpec((NC, BR, Fin), lambda i: (0, i, 0)),
            pl.BlockSpec((BR, Fin), lambda i: (i, 0)),
            pl.BlockSpec((BR, 1), lambda i: (i, 0)),
            pl.BlockSpec((1, Fin), lambda i: (0, 0)),
            pl.BlockSpec((Fin, Fout), lambda i: (0, 0)),
        ],
        out_specs=[
            pl.BlockSpec((BR, Fin), lambda i: (i, 0)),
            pl.BlockSpec((BR, Fout), lambda i: (i, 0)),
        ],
        out_shape=[
            jax.ShapeDtypeStruct((N, Fin), jnp.float32),
            jax.ShapeDtypeStruct((N, Fout), jnp.float32),
        ],
    )(p, g, dinv, b, Wn)


def _tc_final(p, g, dinv, b, F):
    def body(p_ref, g_ref, dinv_ref, b_ref, h_ref):
        h_ref[...] = (dinv_ref[...] * (p_ref[0] + p_ref[1] + g_ref[...])
                      + b_ref[...])

    return pl.pallas_call(
        body,
        grid=(GRID,),
        in_specs=[
            pl.BlockSpec((NC, BR, F), lambda i: (0, i, 0)),
            pl.BlockSpec((BR, F), lambda i: (i, 0)),
            pl.BlockSpec((BR, 1), lambda i: (i, 0)),
            pl.BlockSpec((1, F), lambda i: (0, 0)),
        ],
        out_specs=pl.BlockSpec((BR, F), lambda i: (i, 0)),
        out_shape=jax.ShapeDtypeStruct((N, F), jnp.float32),
    )(p, g, dinv, b)


def kernel(x, train_pos_edge_index, pos_edge_index, neg_edge_index,
           W1, b1, W2, b2, W3, b3):
    src = train_pos_edge_index[0]
    dst = train_pos_edge_index[1]

    deg0, deg1 = _deg_partials(dst)
    degp = jnp.stack([deg0, deg1], axis=1)
    dinv, g1s = _tc_first(degp, x, W1)

    pa = _agg_partials(g1s[0], src, dst, 64)
    pb = _agg_partials(g1s[1], src, dst, 64)
    g2 = _tc_layer1(pa, pb, g1s[0], g1s[1], dinv, b1.reshape(1, -1), W2)

    pb = _agg_partials(g2, src, dst, 64)
    h2, g3 = _tc_layer(pb, g2, dinv, b2.reshape(1, -1), W3, 64, 64)

    pc = _agg_partials(g3, src, dst, 64)
    h3 = _tc_final(pc, g3, dinv, b3.reshape(1, -1), 64)

    ei = jnp.concatenate([pos_edge_index, neg_edge_index], axis=1)
    scores = _edge_scores(h3, ei[0], ei[1])
    return (scores, h2)


# speedup vs baseline: 1.0154x; 1.0154x over previous
import functools

import jax
import jax.numpy as jnp
from jax import lax
from jax.experimental import pallas as pl
from jax.experimental.pallas import tpu as pltpu
from jax.experimental.pallas import tpu_sc as plsc

N = 10000
E = 320000
NC = 2
NS = 16
NW = NC * NS
CHUNK = 80
EPT = E // NW
NCHUNK = EPT // CHUNK
ROWS_T = 624
ROWS_LAST = N - (NS - 1) * ROWS_T
NB = 5


def _mesh():
    return plsc.VectorSubcoreMesh(core_axis_name="c", subcore_axis_name="s")


_SC_PARAMS = pltpu.CompilerParams(use_tc_tiling_on_sc=False)
_SC_PARAMS_NOLAYOUT = pltpu.CompilerParams(use_tc_tiling_on_sc=False,
                                           needs_layout_passes=False)


def _zero_acc(obuf, acc, s, nrow_cols):
    r0 = s * ROWS_T
    if len(nrow_cols) == 1:
        def zrow(i, carry):
            obuf[pl.ds(i * 16, 16)] = jnp.zeros((16,), jnp.float32)
            return carry
        lax.fori_loop(0, ROWS_LAST // 16, zrow, None)
    else:
        F = nrow_cols[1]

        def zrow(i, carry):
            for j in range(F // 16):
                obuf[i, pl.ds(j * 16, 16)] = jnp.zeros((16,), jnp.float32)
            return carry
        lax.fori_loop(0, ROWS_LAST, zrow, None)

    @pl.when(s < NS - 1)
    def _z0():
        pltpu.sync_copy(obuf.at[pl.ds(0, ROWS_T)], acc.at[pl.ds(r0, ROWS_T)])

    @pl.when(s == NS - 1)
    def _z1():
        pltpu.sync_copy(obuf, acc.at[pl.ds(r0, ROWS_LAST)])


def _writeout(obuf, acc, oref, s):
    r0 = s * ROWS_T

    @pl.when(s < NS - 1)
    def _w0():
        pltpu.sync_copy(acc.at[pl.ds(r0, ROWS_T)], obuf.at[pl.ds(0, ROWS_T)])
        pltpu.sync_copy(obuf.at[pl.ds(0, ROWS_T)], oref.at[pl.ds(r0, ROWS_T)])

    @pl.when(s == NS - 1)
    def _w1():
        pltpu.sync_copy(acc.at[pl.ds(r0, ROWS_LAST)], obuf)
        pltpu.sync_copy(obuf, oref.at[pl.ds(r0, ROWS_LAST)])


def _deg_partials(dst):
    @functools.partial(
        pl.kernel,
        mesh=_mesh(),
        compiler_params=_SC_PARAMS_NOLAYOUT,
        out_type=[jax.ShapeDtypeStruct((N,), jnp.float32),
                  jax.ShapeDtypeStruct((N,), jnp.float32)],
        scratch_types=[
            pltpu.VMEM((EPT,), jnp.int32),
            pltpu.VMEM((N,), jnp.float32),
            pltpu.VMEM((NS, ROWS_LAST), jnp.float32),
            pltpu.VMEM((ROWS_LAST,), jnp.float32),
            pltpu.VMEM_SHARED((NS, N), jnp.float32),
        ],
    )
    def k(dst_hbm, out0_hbm, out1_hbm, didx, hist, buf, sbuf, slab):
        c = lax.axis_index("c")
        s = lax.axis_index("s")
        wid = c * NS + s
        pltpu.sync_copy(dst_hbm.at[pl.ds(wid * EPT, EPT)], didx)

        def zh(i, carry):
            hist[pl.ds(i * 16, 16)] = jnp.zeros((16,), jnp.float32)
            return carry

        lax.fori_loop(0, N // 16, zh, None)
        ones16 = jnp.ones((16,), jnp.float32)

        def hb(i, carry):
            idxv = didx[pl.ds(i * 16, 16)]
            plsc.addupdate_scatter(hist, [idxv], ones16)
            return carry

        lax.fori_loop(0, EPT // 16, hb, None)
        pltpu.sync_copy(hist, slab.at[s])
        plsc.subcore_barrier()

        r0 = s * ROWS_T

        @pl.when(s < NS - 1)
        def _l0():
            for r in range(NS):
                pltpu.sync_copy(slab.at[r, pl.ds(r0, ROWS_T)],
                                buf.at[r, pl.ds(0, ROWS_T)])

        @pl.when(s == NS - 1)
        def _l1():
            for r in range(NS):
                pltpu.sync_copy(slab.at[r, pl.ds(r0, ROWS_LAST)], buf.at[r])

        def sb(kk, carry):
            acc16 = buf[0, pl.ds(kk * 16, 16)]
            for r in range(1, NS):
                acc16 = acc16 + buf[r, pl.ds(kk * 16, 16)]
            sbuf[pl.ds(kk * 16, 16)] = acc16
            return carry

        lax.fori_loop(0, ROWS_LAST // 16, sb, None)

        for ci, oref in ((0, out0_hbm), (1, out1_hbm)):
            @pl.when((c == ci) & (s < NS - 1))
            def _w0(oref=oref):
                pltpu.sync_copy(sbuf.at[pl.ds(0, ROWS_T)],
                                oref.at[pl.ds(r0, ROWS_T)])

            @pl.when((c == ci) & (s == NS - 1))
            def _w1(oref=oref):
                pltpu.sync_copy(sbuf, oref.at[pl.ds(r0, ROWS_LAST)])

    return k(dst)


def _agg_partials(g, src, dst, F):
    @functools.partial(
        pl.kernel,
        mesh=_mesh(),
        compiler_params=_SC_PARAMS,
        out_type=jax.ShapeDtypeStruct((NC, N, F), jnp.float32),
        scratch_types=[
            pltpu.VMEM((EPT,), jnp.int32),
            pltpu.VMEM((EPT,), jnp.int32),
            [pltpu.VMEM((CHUNK, F), jnp.float32)] * NB,
            pltpu.VMEM((ROWS_LAST, F), jnp.float32),
            pltpu.VMEM_SHARED((N, F), jnp.float32),
            [pltpu.SemaphoreType.DMA] * NB,
            [pltpu.SemaphoreType.DMA] * NB,
        ],
    )
    def k(g_hbm, src_hbm, dst_hbm, out_hbm, sidx, didx, rows, obuf,
          acc, gsems, ssems):
        c = lax.axis_index("c")
        s = lax.axis_index("s")
        wid = c * NS + s
        pltpu.sync_copy(src_hbm.at[pl.ds(wid * EPT, EPT)], sidx)
        pltpu.sync_copy(dst_hbm.at[pl.ds(wid * EPT, EPT)], didx)
        _zero_acc(obuf, acc, s, (N, F))
        plsc.subcore_barrier()

        def sl(ref, i):
            return ref.at[pl.ds(i * CHUNK, CHUNK)]

        for b in range(NB):
            pltpu.async_copy(g_hbm.at[sl(sidx, b)], rows[b], gsems[b])

        def body(t, carry):
            for b in range(NB):
                i = t * NB + b
                pltpu.make_async_copy(g_hbm.at[sl(sidx, i)], rows[b],
                                      gsems[b]).wait()
                pltpu.async_copy(rows[b], acc.at[sl(didx, i)], ssems[b],
                                 add=True)
                pltpu.make_async_copy(rows[b], acc.at[sl(didx, i)],
                                      ssems[b]).wait()
                pltpu.async_copy(g_hbm.at[sl(sidx, i + NB)], rows[b], gsems[b])
            return carry

        lax.fori_loop(0, NCHUNK // NB - 1, body, None)
        for b in range(NB):
            i = NCHUNK - NB + b
            pltpu.make_async_copy(g_hbm.at[sl(sidx, i)], rows[b],
                                  gsems[b]).wait()
            pltpu.async_copy(rows[b], acc.at[sl(didx, i)], ssems[b], add=True)
            pltpu.make_async_copy(rows[b], acc.at[sl(didx, i)],
                                  ssems[b]).wait()
        plsc.subcore_barrier()
        _writeout(obuf, acc, out_hbm.at[c], s)

    return k(g, src, dst)


def _edge_scores(h3, ei, ej):
    @functools.partial(
        pl.kernel,
        mesh=_mesh(),
        compiler_params=_SC_PARAMS_NOLAYOUT,
        out_type=jax.ShapeDtypeStruct((E,), jnp.float32),
        scratch_types=[
            pltpu.VMEM((EPT,), jnp.int32),
            pltpu.VMEM((EPT,), jnp.int32),
            [pltpu.VMEM((CHUNK, 64), jnp.bfloat16)] * NB,
            [pltpu.VMEM((CHUNK, 64), jnp.bfloat16)] * NB,
            [pltpu.VMEM((CHUNK,), jnp.float32)] * NB,
            pltpu.VMEM((16, 17), jnp.float32),
            [pltpu.SemaphoreType.DMA] * NB,
            [pltpu.SemaphoreType.DMA] * NB,
        ],
    )
    def k(h_hbm, ei_hbm, ej_hbm, out_hbm, ii, jj, ri, rj, sv, pbuf,
          gsems, wsems):
        c = lax.axis_index("c")
        s = lax.axis_index("s")
        wid = c * NS + s
        base = wid * EPT
        pltpu.sync_copy(ei_hbm.at[pl.ds(wid * EPT, EPT)], ii)
        pltpu.sync_copy(ej_hbm.at[pl.ds(wid * EPT, EPT)], jj)

        def sl(ref, i):
            return ref.at[pl.ds(i * CHUNK, CHUNK)]

        def start_gathers(i, b):
            pltpu.async_copy(h_hbm.at[sl(ii, i)], ri[b], gsems[b])
            pltpu.async_copy(h_hbm.at[sl(jj, i)], rj[b], gsems[b])

        def wait_gathers(i, b):
            pltpu.make_async_copy(h_hbm.at[sl(ii, i)], ri[b], gsems[b]).wait()
            pltpu.make_async_copy(h_hbm.at[sl(jj, i)], rj[b], gsems[b]).wait()

        def compute(b):
            def grp_body(grp, carry):
                def e_body(u4, carry2):
                    for u in range(4):
                        e = grp * 16 + u4 * 4 + u
                        p = None
                        for f in range(2):
                            a = plsc.unpack(ri[b][e, pl.ds(f * 32, 32)],
                                            format=plsc.PackFormat.INTERLEAVED)
                            bb = plsc.unpack(rj[b][e, pl.ds(f * 32, 32)],
                                             format=plsc.PackFormat.INTERLEAVED)
                            q = a[0] * bb[0] + a[1] * bb[1]
                            p = q if p is None else p + q
                        pbuf[u4 * 4 + u, pl.ds(0, 16)] = p
                    return carry2

                lax.fori_loop(0, 4, e_body, None)
                rvec = jnp.zeros((16,), jnp.float32)
                urow = lax.iota(jnp.int32, 16)
                for f in range(16):
                    cf = jnp.full((16,), f, jnp.int32)
                    rvec = rvec + plsc.load_gather(pbuf, [urow, cf])
                sv[b][pl.ds(grp * 16, 16)] = rvec
                return carry

            lax.fori_loop(0, CHUNK // 16, grp_body, None)

        def start_write(i, b):
            pltpu.async_copy(sv[b], out_hbm.at[pl.ds(base + i * CHUNK, CHUNK)],
                             wsems[b])

        def wait_write(i, b):
            pltpu.make_async_copy(sv[b],
                                  out_hbm.at[pl.ds(base + i * CHUNK, CHUNK)],
                                  wsems[b]).wait()

        for b in range(NB):
            start_gathers(b, b)

        def body(t, carry):
            for b in range(NB):
                i = t * NB + b
                wait_gathers(i, b)

                @pl.when(t > 0)
                def _ww():
                    wait_write(i - NB, b)

                compute(b)
                start_write(i, b)
                start_gathers(i + NB, b)
            return carry

        lax.fori_loop(0, NCHUNK // NB - 1, body, None)
        for b in range(NB):
            i = NCHUNK - NB + b
            wait_gathers(i, b)
            wait_write(i - NB, b)
            compute(b)
            start_write(i, b)
        for b in range(NB):
            wait_write(NCHUNK - NB + b, b)

    return k(h3, ei, ej)


BR = 2000
GRID = N // BR


def _tc_first(degp, x, W1):
    def body(degp_ref, x_ref, w_ref, dinv_ref, g_ref):
        deg = degp_ref[:, 0:1] + degp_ref[:, 1:2] + 1.0
        dinv = lax.rsqrt(deg)
        dinv_ref[...] = dinv
        g = dinv * jnp.dot(x_ref[...], w_ref[...],
                           preferred_element_type=jnp.float32)
        g_ref[0] = g[:, :64]
        g_ref[1] = g[:, 64:]

    return pl.pallas_call(
        body,
        grid=(GRID,),
        in_specs=[
            pl.BlockSpec((BR, 2), lambda i: (i, 0)),
            pl.BlockSpec((BR, 128), lambda i: (i, 0)),
            pl.BlockSpec((128, 128), lambda i: (0, 0)),
        ],
        out_specs=[
            pl.BlockSpec((BR, 1), lambda i: (i, 0)),
            pl.BlockSpec((2, BR, 64), lambda i: (0, i, 0)),
        ],
        out_shape=[
            jax.ShapeDtypeStruct((N, 1), jnp.float32),
            jax.ShapeDtypeStruct((2, N, 64), jnp.float32),
        ],
    )(degp, x, W1)


def _tc_layer1(pa, pb, ga, gb, dinv, b, Wn):
    def body(pa_ref, pb_ref, ga_ref, gb_ref, dinv_ref, b_ref, w_ref, gn_ref):
        sa = pa_ref[0] + pa_ref[1] + ga_ref[...]
        sb = pb_ref[0] + pb_ref[1] + gb_ref[...]
        s = jnp.concatenate([sa, sb], axis=1)
        h = jnp.maximum(dinv_ref[...] * s + b_ref[...], 0.0)
        gn_ref[...] = dinv_ref[...] * jnp.dot(h, w_ref[...],
                                              preferred_element_type=jnp.float32)

    return pl.pallas_call(
        body,
        grid=(GRID,),
        in_specs=[
            pl.BlockSpec((NC, BR, 64), lambda i: (0, i, 0)),
            pl.BlockSpec((NC, BR, 64), lambda i: (0, i, 0)),
            pl.BlockSpec((BR, 64), lambda i: (i, 0)),
            pl.BlockSpec((BR, 64), lambda i: (i, 0)),
            pl.BlockSpec((BR, 1), lambda i: (i, 0)),
            pl.BlockSpec((1, 128), lambda i: (0, 0)),
            pl.BlockSpec((128, 64), lambda i: (0, 0)),
        ],
        out_specs=pl.BlockSpec((BR, 64), lambda i: (i, 0)),
        out_shape=jax.ShapeDtypeStruct((N, 64), jnp.float32),
    )(pa, pb, ga, gb, dinv, b, Wn)


def _tc_layer(p, g, dinv, b, Wn, Fin, Fout):
    def body(p_ref, g_ref, dinv_ref, b_ref, w_ref, h_ref, gn_ref):
        pr = p_ref[0] + p_ref[1]
        h = jnp.maximum(dinv_ref[...] * (pr + g_ref[...]) + b_ref[...], 0.0)
        h_ref[...] = h
        gn_ref[...] = dinv_ref[...] * jnp.dot(h, w_ref[...],
                                              preferred_element_type=jnp.float32)

    return pl.pallas_call(
        body,
        grid=(GRID,),
        in_specs=[
            pl.BlockSpec((NC, BR, Fin), lambda i: (0, i, 0)),
            pl.BlockSpec((BR, Fin), lambda i: (i, 0)),
            pl.BlockSpec((BR, 1), lambda i: (i, 0)),
            pl.BlockSpec((1, Fin), lambda i: (0, 0)),
            pl.BlockSpec((Fin, Fout), lambda i: (0, 0)),
        ],
        out_specs=[
            pl.BlockSpec((BR, Fin), lambda i: (i, 0)),
            pl.BlockSpec((BR, Fout), lambda i: (i, 0)),
        ],
        out_shape=[
            jax.ShapeDtypeStruct((N, Fin), jnp.float32),
            jax.ShapeDtypeStruct((N, Fout), jnp.float32),
        ],
    )(p, g, dinv, b, Wn)


def _tc_final(p, g, dinv, b, F):
    def body(p_ref, g_ref, dinv_ref, b_ref, h_ref):
        h_ref[...] = (dinv_ref[...] * (p_ref[0] + p_ref[1] + g_ref[...])
                      + b_ref[...]).astype(jnp.bfloat16)

    return pl.pallas_call(
        body,
        grid=(GRID,),
        in_specs=[
            pl.BlockSpec((NC, BR, F), lambda i: (0, i, 0)),
            pl.BlockSpec((BR, F), lambda i: (i, 0)),
            pl.BlockSpec((BR, 1), lambda i: (i, 0)),
            pl.BlockSpec((1, F), lambda i: (0, 0)),
        ],
        out_specs=pl.BlockSpec((BR, F), lambda i: (i, 0)),
        out_shape=jax.ShapeDtypeStruct((N, F), jnp.bfloat16),
    )(p, g, dinv, b)


def kernel(x, train_pos_edge_index, pos_edge_index, neg_edge_index,
           W1, b1, W2, b2, W3, b3):
    src = train_pos_edge_index[0]
    dst = train_pos_edge_index[1]

    deg0, deg1 = _deg_partials(dst)
    degp = jnp.stack([deg0, deg1], axis=1)
    dinv, g1s = _tc_first(degp, x, W1)

    pa = _agg_partials(g1s[0], src, dst, 64)
    pb = _agg_partials(g1s[1], src, dst, 64)
    g2 = _tc_layer1(pa, pb, g1s[0], g1s[1], dinv, b1.reshape(1, -1), W2)

    pb = _agg_partials(g2, src, dst, 64)
    h2, g3 = _tc_layer(pb, g2, dinv, b2.reshape(1, -1), W3, 64, 64)

    pc = _agg_partials(g3, src, dst, 64)
    h3 = _tc_final(pc, g3, dinv, b3.reshape(1, -1), 64)

    ei = jnp.concatenate([pos_edge_index, neg_edge_index], axis=1)
    scores = _edge_scores(h3, ei[0], ei[1])
    return (scores, h2)


# speedup vs baseline: 1.0228x; 1.0073x over previous
import functools

import jax
import jax.numpy as jnp
from jax import lax
from jax.experimental import pallas as pl
from jax.experimental.pallas import tpu as pltpu
from jax.experimental.pallas import tpu_sc as plsc

N = 10000
E = 320000
NC = 2
NS = 16
NW = NC * NS
CHUNK = 80
EPT = E // NW
NCHUNK = EPT // CHUNK
ROWS_T = 624
ROWS_LAST = N - (NS - 1) * ROWS_T
NB = 5


def _mesh():
    return plsc.VectorSubcoreMesh(core_axis_name="c", subcore_axis_name="s")


_SC_PARAMS = pltpu.CompilerParams(use_tc_tiling_on_sc=False)
_SC_PARAMS_NOLAYOUT = pltpu.CompilerParams(use_tc_tiling_on_sc=False,
                                           needs_layout_passes=False)


def _zero_acc(obuf, acc, s, nrow_cols):
    r0 = s * ROWS_T
    if len(nrow_cols) == 1:
        def zrow(i, carry):
            obuf[pl.ds(i * 16, 16)] = jnp.zeros((16,), jnp.float32)
            return carry
        lax.fori_loop(0, ROWS_LAST // 16, zrow, None)
    else:
        F = nrow_cols[1]

        def zrow(i, carry):
            for j in range(F // 16):
                obuf[i, pl.ds(j * 16, 16)] = jnp.zeros((16,), jnp.float32)
            return carry
        lax.fori_loop(0, ROWS_LAST, zrow, None)

    @pl.when(s < NS - 1)
    def _z0():
        pltpu.sync_copy(obuf.at[pl.ds(0, ROWS_T)], acc.at[pl.ds(r0, ROWS_T)])

    @pl.when(s == NS - 1)
    def _z1():
        pltpu.sync_copy(obuf, acc.at[pl.ds(r0, ROWS_LAST)])


def _writeout(obuf, acc, oref, s):
    r0 = s * ROWS_T

    @pl.when(s < NS - 1)
    def _w0():
        pltpu.sync_copy(acc.at[pl.ds(r0, ROWS_T)], obuf.at[pl.ds(0, ROWS_T)])
        pltpu.sync_copy(obuf.at[pl.ds(0, ROWS_T)], oref.at[pl.ds(r0, ROWS_T)])

    @pl.when(s == NS - 1)
    def _w1():
        pltpu.sync_copy(acc.at[pl.ds(r0, ROWS_LAST)], obuf)
        pltpu.sync_copy(obuf, oref.at[pl.ds(r0, ROWS_LAST)])


def _deg_partials(dst):
    @functools.partial(
        pl.kernel,
        mesh=_mesh(),
        compiler_params=_SC_PARAMS_NOLAYOUT,
        out_type=[jax.ShapeDtypeStruct((N,), jnp.float32),
                  jax.ShapeDtypeStruct((N,), jnp.float32)],
        scratch_types=[
            pltpu.VMEM((EPT,), jnp.int32),
            pltpu.VMEM((N,), jnp.float32),
            pltpu.VMEM((NS, ROWS_LAST), jnp.float32),
            pltpu.VMEM((ROWS_LAST,), jnp.float32),
            pltpu.VMEM_SHARED((NS, N), jnp.float32),
        ],
    )
    def k(dst_hbm, out0_hbm, out1_hbm, didx, hist, buf, sbuf, slab):
        c = lax.axis_index("c")
        s = lax.axis_index("s")
        wid = c * NS + s
        pltpu.sync_copy(dst_hbm.at[pl.ds(wid * EPT, EPT)], didx)

        def zh(i, carry):
            hist[pl.ds(i * 16, 16)] = jnp.zeros((16,), jnp.float32)
            return carry

        lax.fori_loop(0, N // 16, zh, None)
        ones16 = jnp.ones((16,), jnp.float32)

        def hb(i, carry):
            idxv = didx[pl.ds(i * 16, 16)]
            plsc.addupdate_scatter(hist, [idxv], ones16)
            return carry

        lax.fori_loop(0, EPT // 16, hb, None)
        pltpu.sync_copy(hist, slab.at[s])
        plsc.subcore_barrier()

        r0 = s * ROWS_T

        @pl.when(s < NS - 1)
        def _l0():
            for r in range(NS):
                pltpu.sync_copy(slab.at[r, pl.ds(r0, ROWS_T)],
                                buf.at[r, pl.ds(0, ROWS_T)])

        @pl.when(s == NS - 1)
        def _l1():
            for r in range(NS):
                pltpu.sync_copy(slab.at[r, pl.ds(r0, ROWS_LAST)], buf.at[r])

        def sb(kk, carry):
            acc16 = buf[0, pl.ds(kk * 16, 16)]
            for r in range(1, NS):
                acc16 = acc16 + buf[r, pl.ds(kk * 16, 16)]
            sbuf[pl.ds(kk * 16, 16)] = acc16
            return carry

        lax.fori_loop(0, ROWS_LAST // 16, sb, None)

        for ci, oref in ((0, out0_hbm), (1, out1_hbm)):
            @pl.when((c == ci) & (s < NS - 1))
            def _w0(oref=oref):
                pltpu.sync_copy(sbuf.at[pl.ds(0, ROWS_T)],
                                oref.at[pl.ds(r0, ROWS_T)])

            @pl.when((c == ci) & (s == NS - 1))
            def _w1(oref=oref):
                pltpu.sync_copy(sbuf, oref.at[pl.ds(r0, ROWS_LAST)])

    return k(dst)


def _agg_partials(g, src, dst, F):
    @functools.partial(
        pl.kernel,
        mesh=_mesh(),
        compiler_params=_SC_PARAMS,
        out_type=jax.ShapeDtypeStruct((NC, N, F), jnp.float32),
        scratch_types=[
            pltpu.VMEM((EPT,), jnp.int32),
            pltpu.VMEM((EPT,), jnp.int32),
            [pltpu.VMEM((CHUNK, F), jnp.float32)] * NB,
            pltpu.VMEM((ROWS_LAST, F), jnp.float32),
            pltpu.VMEM_SHARED((N, F), jnp.float32),
            [pltpu.SemaphoreType.DMA] * NB,
            [pltpu.SemaphoreType.DMA] * NB,
        ],
    )
    def k(g_hbm, src_hbm, dst_hbm, out_hbm, sidx, didx, rows, obuf,
          acc, gsems, ssems):
        c = lax.axis_index("c")
        s = lax.axis_index("s")
        wid = c * NS + s
        pltpu.sync_copy(src_hbm.at[pl.ds(wid * EPT, EPT)], sidx)
        pltpu.sync_copy(dst_hbm.at[pl.ds(wid * EPT, EPT)], didx)
        _zero_acc(obuf, acc, s, (N, F))
        plsc.subcore_barrier()

        def sl(ref, i):
            return ref.at[pl.ds(i * CHUNK, CHUNK)]

        for b in range(NB):
            pltpu.async_copy(g_hbm.at[sl(sidx, b)], rows[b], gsems[b])

        def body(t, carry):
            for b in range(NB):
                i = t * NB + b
                pltpu.make_async_copy(g_hbm.at[sl(sidx, i)], rows[b],
                                      gsems[b]).wait()
                pltpu.async_copy(rows[b], acc.at[sl(didx, i)], ssems[b],
                                 add=True)
                pltpu.make_async_copy(rows[b], acc.at[sl(didx, i)],
                                      ssems[b]).wait()
                pltpu.async_copy(g_hbm.at[sl(sidx, i + NB)], rows[b], gsems[b])
            return carry

        lax.fori_loop(0, NCHUNK // NB - 1, body, None)
        for b in range(NB):
            i = NCHUNK - NB + b
            pltpu.make_async_copy(g_hbm.at[sl(sidx, i)], rows[b],
                                  gsems[b]).wait()
            pltpu.async_copy(rows[b], acc.at[sl(didx, i)], ssems[b], add=True)
            pltpu.make_async_copy(rows[b], acc.at[sl(didx, i)],
                                  ssems[b]).wait()
        plsc.subcore_barrier()
        _writeout(obuf, acc, out_hbm.at[c], s)

    return k(g, src, dst)


def _edge_scores(h3, ei, ej):
    @functools.partial(
        pl.kernel,
        mesh=_mesh(),
        compiler_params=_SC_PARAMS_NOLAYOUT,
        out_type=jax.ShapeDtypeStruct((E,), jnp.float32),
        scratch_types=[
            pltpu.VMEM((EPT,), jnp.int32),
            pltpu.VMEM((EPT,), jnp.int32),
            [pltpu.VMEM((CHUNK, 64), jnp.bfloat16)] * NB,
            [pltpu.VMEM((CHUNK, 64), jnp.bfloat16)] * NB,
            [pltpu.VMEM((CHUNK,), jnp.float32)] * NB,
            pltpu.VMEM((16, 17), jnp.float32),
            [pltpu.SemaphoreType.DMA] * NB,
            [pltpu.SemaphoreType.DMA] * NB,
        ],
    )
    def k(h_hbm, ei_hbm, ej_hbm, out_hbm, ii, jj, ri, rj, sv, pbuf,
          gsems, wsems):
        c = lax.axis_index("c")
        s = lax.axis_index("s")
        wid = c * NS + s
        base = wid * EPT
        pltpu.sync_copy(ei_hbm.at[pl.ds(wid * EPT, EPT)], ii)
        pltpu.sync_copy(ej_hbm.at[pl.ds(wid * EPT, EPT)], jj)

        def sl(ref, i):
            return ref.at[pl.ds(i * CHUNK, CHUNK)]

        def start_gathers(i, b):
            pltpu.async_copy(h_hbm.at[sl(ii, i)], ri[b], gsems[b])
            pltpu.async_copy(h_hbm.at[sl(jj, i)], rj[b], gsems[b])

        def wait_gathers(i, b):
            pltpu.make_async_copy(h_hbm.at[sl(ii, i)], ri[b], gsems[b]).wait()
            pltpu.make_async_copy(h_hbm.at[sl(jj, i)], rj[b], gsems[b]).wait()

        def compute(b):
            def grp_body(grp, carry):
                def e_body(u4, carry2):
                    for u in range(4):
                        e = grp * 16 + u4 * 4 + u
                        m = (ri[b][e, pl.ds(0, 32)] * rj[b][e, pl.ds(0, 32)]
                             + ri[b][e, pl.ds(32, 32)]
                             * rj[b][e, pl.ds(32, 32)])
                        a0, a1 = plsc.unpack(
                            m, format=plsc.PackFormat.INTERLEAVED)
                        pbuf[u4 * 4 + u, pl.ds(0, 16)] = a0 + a1
                    return carry2

                lax.fori_loop(0, 4, e_body, None)
                rvec = jnp.zeros((16,), jnp.float32)
                urow = lax.iota(jnp.int32, 16)
                for f in range(16):
                    cf = jnp.full((16,), f, jnp.int32)
                    rvec = rvec + plsc.load_gather(pbuf, [urow, cf])
                sv[b][pl.ds(grp * 16, 16)] = rvec
                return carry

            lax.fori_loop(0, CHUNK // 16, grp_body, None)

        def start_write(i, b):
            pltpu.async_copy(sv[b], out_hbm.at[pl.ds(base + i * CHUNK, CHUNK)],
                             wsems[b])

        def wait_write(i, b):
            pltpu.make_async_copy(sv[b],
                                  out_hbm.at[pl.ds(base + i * CHUNK, CHUNK)],
                                  wsems[b]).wait()

        for b in range(NB):
            start_gathers(b, b)

        def body(t, carry):
            for b in range(NB):
                i = t * NB + b
                wait_gathers(i, b)

                @pl.when(t > 0)
                def _ww():
                    wait_write(i - NB, b)

                compute(b)
                start_write(i, b)
                start_gathers(i + NB, b)
            return carry

        lax.fori_loop(0, NCHUNK // NB - 1, body, None)
        for b in range(NB):
            i = NCHUNK - NB + b
            wait_gathers(i, b)
            wait_write(i - NB, b)
            compute(b)
            start_write(i, b)
        for b in range(NB):
            wait_write(NCHUNK - NB + b, b)

    return k(h3, ei, ej)


BR = 2000
GRID = N // BR


def _tc_first(degp, x, W1):
    def body(degp_ref, x_ref, w_ref, dinv_ref, g_ref):
        deg = degp_ref[:, 0:1] + degp_ref[:, 1:2] + 1.0
        dinv = lax.rsqrt(deg)
        dinv_ref[...] = dinv
        g = dinv * jnp.dot(x_ref[...], w_ref[...],
                           preferred_element_type=jnp.float32)
        g_ref[0] = g[:, :64]
        g_ref[1] = g[:, 64:]

    return pl.pallas_call(
        body,
        grid=(GRID,),
        in_specs=[
            pl.BlockSpec((BR, 2), lambda i: (i, 0)),
            pl.BlockSpec((BR, 128), lambda i: (i, 0)),
            pl.BlockSpec((128, 128), lambda i: (0, 0)),
        ],
        out_specs=[
            pl.BlockSpec((BR, 1), lambda i: (i, 0)),
            pl.BlockSpec((2, BR, 64), lambda i: (0, i, 0)),
        ],
        out_shape=[
            jax.ShapeDtypeStruct((N, 1), jnp.float32),
            jax.ShapeDtypeStruct((2, N, 64), jnp.float32),
        ],
    )(degp, x, W1)


def _tc_layer1(pa, pb, ga, gb, dinv, b, Wn):
    def body(pa_ref, pb_ref, ga_ref, gb_ref, dinv_ref, b_ref, w_ref, gn_ref):
        sa = pa_ref[0] + pa_ref[1] + ga_ref[...]
        sb = pb_ref[0] + pb_ref[1] + gb_ref[...]
        s = jnp.concatenate([sa, sb], axis=1)
        h = jnp.maximum(dinv_ref[...] * s + b_ref[...], 0.0)
        gn_ref[...] = dinv_ref[...] * jnp.dot(h, w_ref[...],
                                              preferred_element_type=jnp.float32)

    return pl.pallas_call(
        body,
        grid=(GRID,),
        in_specs=[
            pl.BlockSpec((NC, BR, 64), lambda i: (0, i, 0)),
            pl.BlockSpec((NC, BR, 64), lambda i: (0, i, 0)),
            pl.BlockSpec((BR, 64), lambda i: (i, 0)),
            pl.BlockSpec((BR, 64), lambda i: (i, 0)),
            pl.BlockSpec((BR, 1), lambda i: (i, 0)),
            pl.BlockSpec((1, 128), lambda i: (0, 0)),
            pl.BlockSpec((128, 64), lambda i: (0, 0)),
        ],
        out_specs=pl.BlockSpec((BR, 64), lambda i: (i, 0)),
        out_shape=jax.ShapeDtypeStruct((N, 64), jnp.float32),
    )(pa, pb, ga, gb, dinv, b, Wn)


def _tc_layer(p, g, dinv, b, Wn, Fin, Fout):
    def body(p_ref, g_ref, dinv_ref, b_ref, w_ref, h_ref, gn_ref):
        pr = p_ref[0] + p_ref[1]
        h = jnp.maximum(dinv_ref[...] * (pr + g_ref[...]) + b_ref[...], 0.0)
        h_ref[...] = h
        gn_ref[...] = dinv_ref[...] * jnp.dot(h, w_ref[...],
                                              preferred_element_type=jnp.float32)

    return pl.pallas_call(
        body,
        grid=(GRID,),
        in_specs=[
            pl.BlockSpec((NC, BR, Fin), lambda i: (0, i, 0)),
            pl.BlockSpec((BR, Fin), lambda i: (i, 0)),
            pl.BlockSpec((BR, 1), lambda i: (i, 0)),
            pl.BlockSpec((1, Fin), lambda i: (0, 0)),
            pl.BlockSpec((Fin, Fout), lambda i: (0, 0)),
        ],
        out_specs=[
            pl.BlockSpec((BR, Fin), lambda i: (i, 0)),
            pl.BlockSpec((BR, Fout), lambda i: (i, 0)),
        ],
        out_shape=[
            jax.ShapeDtypeStruct((N, Fin), jnp.float32),
            jax.ShapeDtypeStruct((N, Fout), jnp.float32),
        ],
    )(p, g, dinv, b, Wn)


def _tc_final(p, g, dinv, b, F):
    def body(p_ref, g_ref, dinv_ref, b_ref, h_ref):
        h_ref[...] = (dinv_ref[...] * (p_ref[0] + p_ref[1] + g_ref[...])
                      + b_ref[...]).astype(jnp.bfloat16)

    return pl.pallas_call(
        body,
        grid=(GRID,),
        in_specs=[
            pl.BlockSpec((NC, BR, F), lambda i: (0, i, 0)),
            pl.BlockSpec((BR, F), lambda i: (i, 0)),
            pl.BlockSpec((BR, 1), lambda i: (i, 0)),
            pl.BlockSpec((1, F), lambda i: (0, 0)),
        ],
        out_specs=pl.BlockSpec((BR, F), lambda i: (i, 0)),
        out_shape=jax.ShapeDtypeStruct((N, F), jnp.bfloat16),
    )(p, g, dinv, b)


def kernel(x, train_pos_edge_index, pos_edge_index, neg_edge_index,
           W1, b1, W2, b2, W3, b3):
    src = train_pos_edge_index[0]
    dst = train_pos_edge_index[1]

    deg0, deg1 = _deg_partials(dst)
    degp = jnp.stack([deg0, deg1], axis=1)
    dinv, g1s = _tc_first(degp, x, W1)

    pa = _agg_partials(g1s[0], src, dst, 64)
    pb = _agg_partials(g1s[1], src, dst, 64)
    g2 = _tc_layer1(pa, pb, g1s[0], g1s[1], dinv, b1.reshape(1, -1), W2)

    pb = _agg_partials(g2, src, dst, 64)
    h2, g3 = _tc_layer(pb, g2, dinv, b2.reshape(1, -1), W3, 64, 64)

    pc = _agg_partials(g3, src, dst, 64)
    h3 = _tc_final(pc, g3, dinv, b3.reshape(1, -1), 64)

    ei = jnp.concatenate([pos_edge_index, neg_edge_index], axis=1)
    scores = _edge_scores(h3, ei[0], ei[1])
    return (scores, h2)
